# baseline (speedup 1.0000x reference)
import functools

import jax
import jax.numpy as jnp
from jax.experimental import pallas as pl
from jax.experimental.pallas import tpu as pltpu

_NCH = 16
_CH = 1024


def _mm_kernel(x_hbm, w_ref, b_ref, o_ref, xbuf, sems):
    def _copy(i):
        return pltpu.make_async_copy(
            x_hbm.at[pl.ds(i * _CH, _CH), :],
            xbuf.at[pl.ds(i * _CH, _CH), :],
            sems.at[i],
        )

    for i in range(_NCH):
        _copy(i).start()
    for i in range(_NCH):
        _copy(i).wait()
    o_ref[...] = (
        jnp.dot(xbuf[...], w_ref[...], preferred_element_type=jnp.float32)
        + b_ref[...]
    )


@functools.partial(jax.jit, static_argnames=())
def kernel(x, W, b):
    B, V = x.shape
    D = W.shape[1]
    b2 = b.reshape(1, D)
    out = pl.pallas_call(
        _mm_kernel,
        in_specs=[
            pl.BlockSpec(memory_space=pltpu.MemorySpace.HBM),
            pl.BlockSpec((V, D), lambda: (0, 0)),
            pl.BlockSpec((1, D), lambda: (0, 0)),
        ],
        out_specs=pl.BlockSpec((B, D), lambda: (0, 0)),
        out_shape=jax.ShapeDtypeStruct((B, D), jnp.float32),
        scratch_shapes=[
            pltpu.VMEM((B, V), jnp.float32),
            pltpu.SemaphoreType.DMA((_NCH,)),
        ],
    )(x, W, b2)
    return out


# speedup vs baseline: 1.0142x; 1.0142x over previous
import functools

import jax
import jax.numpy as jnp
from jax.experimental import pallas as pl
from jax.experimental.pallas import tpu as pltpu

_CH = 2048


def _mm_kernel(x_ref, w_ref, b_ref, o_ref):
    o_ref[...] = (
        jnp.dot(x_ref[...], w_ref[...], preferred_element_type=jnp.float32)
        + b_ref[...]
    )


@functools.partial(jax.jit, static_argnames=())
def kernel(x, W, b):
    B, V = x.shape
    D = W.shape[1]
    b2 = b.reshape(1, D)
    n = B // _CH
    out = pl.pallas_call(
        _mm_kernel,
        grid=(n,),
        in_specs=[
            pl.BlockSpec((_CH, V), lambda i: (i, 0)),
            pl.BlockSpec((V, D), lambda i: (0, 0)),
            pl.BlockSpec((1, D), lambda i: (0, 0)),
        ],
        out_specs=pl.BlockSpec((_CH, D), lambda i: (i, 0)),
        out_shape=jax.ShapeDtypeStruct((B, D), jnp.float32),
        compiler_params=pltpu.CompilerParams(
            dimension_semantics=("parallel",),
        ),
    )(x, W, b2)
    return out
